# Initial kernel scaffold; baseline (speedup 1.0000x reference)
import jax
import jax.numpy as jnp
from jax.experimental import pallas as pl


def kernel(x, edge_index, Wl1, Wr1, b1, Wl2, Wr2, b2):
    raise NotImplementedError("write your pallas kernel here")


# speedup vs baseline: 12.7442x; 12.7442x over previous
import functools

import jax
import jax.numpy as jnp
from jax import lax
from jax.experimental import pallas as pl
from jax.experimental.pallas import tpu as pltpu
from jax.experimental.pallas import tpu_sc as plsc

N = 10000
E = 320000
D_IN = 128
D_HID = 16
D_OUT = 40

NC = 2
NS = 16
NW = NC * NS
B = 128
NB = -(-E // (NW * B))
EPW = NB * B
EPAD = NW * EPW
NPAD = 10112
RPT = NPAD // NS


def _make_seg_sum(F: int):
    mesh = plsc.VectorSubcoreMesh(core_axis_name="c", subcore_axis_name="s")

    @functools.partial(
        pl.kernel,
        out_type=jax.ShapeDtypeStruct((NC, NPAD, F), jnp.float32),
        mesh=mesh,
        scratch_types=[
            pltpu.VMEM((NB, B), jnp.int32),
            pltpu.VMEM((NB, B), jnp.int32),
            pltpu.VMEM((B, F), jnp.float32),
            pltpu.VMEM((RPT, F), jnp.float32),
            pltpu.VMEM_SHARED((NPAD, F), jnp.float32),
            pltpu.SemaphoreType.DMA,
        ],
        compiler_params=pltpu.CompilerParams(use_tc_tiling_on_sc=False),
    )
    def seg_sum(table, src, dst, out, src_v, dst_v, rows_v, zer_v, acc, sem):
        c = lax.axis_index("c")
        s = lax.axis_index("s")
        wid = c * NS + s

        def zero_body(i, _):
            for j in range(F // 16):
                zer_v[i, pl.ds(j * 16, 16)] = jnp.zeros((16,), jnp.float32)
            return 0

        lax.fori_loop(0, RPT, zero_body, 0)
        pltpu.sync_copy(zer_v, acc.at[pl.ds(s * RPT, RPT)])
        plsc.subcore_barrier()

        pltpu.sync_copy(src.at[wid], src_v)
        pltpu.sync_copy(dst.at[wid], dst_v)

        def batch_body(j, _):
            pltpu.async_copy(table.at[src_v.at[j]], rows_v, sem).wait()
            pltpu.sync_copy(rows_v, acc.at[dst_v.at[j]], add=True)
            return 0

        lax.fori_loop(0, NB, batch_body, 0)
        plsc.subcore_barrier()

        pltpu.sync_copy(acc.at[pl.ds(s * RPT, RPT)],
                        out.at[c, pl.ds(s * RPT, RPT)])

    return seg_sum


_seg_sum32 = _make_seg_sum(32)
_seg_sum16 = _make_seg_sum(16)


def _k1_body(x_ref, w_ref, oh_ref, t1_ref, r1_ref):
    pc = jnp.dot(x_ref[:, :], w_ref[:, :], preferred_element_type=jnp.float32)
    t1_ref[:, :] = pc[:, 0:32] + oh_ref[:, :]
    r1_ref[:, :] = pc[:, 32:48]


def _k2_body(acc_ref, r1_ref, b1_ref, h_ref, dinv_ref):
    s1 = acc_ref[0, :N, 0:16] + acc_ref[1, :N, 0:16]
    deg = acc_ref[0, :N, 16:17] + acc_ref[1, :N, 16:17]
    dinv = 1.0 / jnp.maximum(deg, 1.0)
    h_ref[:, :] = jnp.maximum(s1 * dinv + b1_ref[:, :] + r1_ref[:, :], 0.0)
    dinv_ref[:, :] = dinv


def _k3_body(acc_ref, h_ref, dinv_ref, w2_ref, b2_ref, out_ref):
    s2 = acc_ref[0, :N, :] + acc_ref[1, :N, :]
    a = s2 * dinv_ref[:, :]
    cat = jnp.concatenate([a, h_ref[:, :]], axis=1)
    z = jnp.dot(cat, w2_ref[:, :], preferred_element_type=jnp.float32)
    z = z + b2_ref[:, :]
    m = jnp.max(z, axis=1, keepdims=True)
    e = jnp.exp(z - m)
    lse = jnp.log(jnp.sum(e, axis=1, keepdims=True))
    out_ref[:, :] = z - m - lse


def kernel(x, edge_index, Wl1, Wr1, b1, Wl2, Wr2, b2):
    src = edge_index[0].astype(jnp.int32)
    dst = edge_index[1].astype(jnp.int32)
    pad = EPAD - E
    src_p = jnp.concatenate([src, jnp.zeros((pad,), jnp.int32)])
    dst_p = jnp.concatenate([dst, jnp.full((pad,), N, jnp.int32)])
    src_r = src_p.reshape(NW, NB, B)
    dst_r = dst_p.reshape(NW, NB, B)

    w = jnp.concatenate(
        [Wl1.T, jnp.zeros((D_IN, 16), jnp.float32), Wr1.T], axis=1)
    oh = jnp.zeros((1, 32), jnp.float32).at[0, 16].set(1.0)
    t1, r1 = pl.pallas_call(
        _k1_body,
        out_shape=[
            jax.ShapeDtypeStruct((N, 32), jnp.float32),
            jax.ShapeDtypeStruct((N, 16), jnp.float32),
        ],
    )(x, w, oh)

    acc1 = _seg_sum32(t1, src_r, dst_r)

    h, dinv = pl.pallas_call(
        _k2_body,
        out_shape=[
            jax.ShapeDtypeStruct((N, 16), jnp.float32),
            jax.ShapeDtypeStruct((N, 1), jnp.float32),
        ],
    )(acc1, r1, b1.reshape(1, 16))

    acc2 = _seg_sum16(h, src_r, dst_r)

    w2 = jnp.concatenate([Wl2.T, Wr2.T], axis=0)
    out = pl.pallas_call(
        _k3_body,
        out_shape=jax.ShapeDtypeStruct((N, D_OUT), jnp.float32),
    )(acc2, h, dinv, w2, b2.reshape(1, D_OUT))
    return out


# speedup vs baseline: 13.5396x; 1.0624x over previous
import functools

import jax
import jax.numpy as jnp
from jax import lax
from jax.experimental import pallas as pl
from jax.experimental.pallas import tpu as pltpu
from jax.experimental.pallas import tpu_sc as plsc

N = 10000
E = 320000
D_IN = 128
D_HID = 16
D_OUT = 40

NC = 2
NS = 16
NW = NC * NS
B = 128
NBUF = 8
PIPE = 4
NB = 80
EPW = NB * B
EPAD = NW * EPW
NPAD = 10112
RPT = NPAD // NS


def _make_seg_sum(F: int):
    mesh = plsc.VectorSubcoreMesh(core_axis_name="c", subcore_axis_name="s")

    @functools.partial(
        pl.kernel,
        out_type=jax.ShapeDtypeStruct((NC, NPAD, F), jnp.float32),
        mesh=mesh,
        scratch_types=[
            pltpu.VMEM((NB, B), jnp.int32),
            pltpu.VMEM((NB, B), jnp.int32),
            [pltpu.VMEM((B, F), jnp.float32)] * NBUF,
            pltpu.VMEM((RPT, F), jnp.float32),
            pltpu.VMEM_SHARED((NPAD, F), jnp.float32),
            [pltpu.SemaphoreType.DMA] * NBUF,
            [pltpu.SemaphoreType.DMA] * NBUF,
        ],
        compiler_params=pltpu.CompilerParams(use_tc_tiling_on_sc=False),
    )
    def seg_sum(table, src, dst, out, src_v, dst_v, rows, zer_v, acc,
                gsem, ssem):
        c = lax.axis_index("c")
        s = lax.axis_index("s")
        wid = c * NS + s

        def zero_body(i, _):
            for j in range(F // 16):
                zer_v[i, pl.ds(j * 16, 16)] = jnp.zeros((16,), jnp.float32)
            return 0

        lax.fori_loop(0, RPT, zero_body, 0)
        pltpu.sync_copy(zer_v, acc.at[pl.ds(s * RPT, RPT)])
        plsc.subcore_barrier()

        pltpu.sync_copy(src.at[wid], src_v)
        pltpu.sync_copy(dst.at[wid], dst_v)

        for b in range(PIPE):
            pltpu.async_copy(table.at[src_v.at[b]], rows[b], gsem[b])

        def group(g, _):
            j0 = g * NBUF
            for b in range(NBUF):
                j = j0 + b
                pltpu.make_async_copy(
                    table.at[src_v.at[j]], rows[b], gsem[b]).wait()
                pltpu.async_copy(rows[b], acc.at[dst_v.at[j]], ssem[b],
                                 add=True)
                b2 = (b + PIPE) % NBUF
                jn = j + PIPE
                prev = jn - NBUF

                @pl.when(jn < NB)
                def _():
                    @pl.when(prev >= 0)
                    def _():
                        pltpu.make_async_copy(
                            rows[b2], acc.at[dst_v.at[prev]],
                            ssem[b2]).wait()
                    pltpu.async_copy(table.at[src_v.at[jn]], rows[b2],
                                     gsem[b2])
            return 0

        lax.fori_loop(0, NB // NBUF, group, 0)
        for b in range(NBUF):
            pltpu.make_async_copy(
                rows[b], acc.at[dst_v.at[NB - NBUF + b]], ssem[b]).wait()
        plsc.subcore_barrier()

        pltpu.sync_copy(acc.at[pl.ds(s * RPT, RPT)],
                        out.at[c, pl.ds(s * RPT, RPT)])

    return seg_sum


_seg_sum32 = _make_seg_sum(32)
_seg_sum16 = _make_seg_sum(16)


def _k1_body(x_ref, w_ref, oh_ref, t1_ref, r1_ref):
    pc = jnp.dot(x_ref[:, :], w_ref[:, :], preferred_element_type=jnp.float32)
    t1_ref[:, :] = pc[:, 0:32] + oh_ref[:, :]
    r1_ref[:, :] = pc[:, 32:48]


def _k2_body(acc_ref, r1_ref, b1_ref, h_ref, dinv_ref):
    s1 = acc_ref[0, :N, 0:16] + acc_ref[1, :N, 0:16]
    deg = acc_ref[0, :N, 16:17] + acc_ref[1, :N, 16:17]
    dinv = 1.0 / jnp.maximum(deg, 1.0)
    h_ref[:, :] = jnp.maximum(s1 * dinv + b1_ref[:, :] + r1_ref[:, :], 0.0)
    dinv_ref[:, :] = dinv


def _k3_body(acc_ref, h_ref, dinv_ref, w2_ref, b2_ref, out_ref):
    s2 = acc_ref[0, :N, :] + acc_ref[1, :N, :]
    a = s2 * dinv_ref[:, :]
    cat = jnp.concatenate([a, h_ref[:, :]], axis=1)
    z = jnp.dot(cat, w2_ref[:, :], preferred_element_type=jnp.float32)
    z = z + b2_ref[:, :]
    m = jnp.max(z, axis=1, keepdims=True)
    e = jnp.exp(z - m)
    lse = jnp.log(jnp.sum(e, axis=1, keepdims=True))
    out_ref[:, :] = z - m - lse


def kernel(x, edge_index, Wl1, Wr1, b1, Wl2, Wr2, b2):
    src = edge_index[0].astype(jnp.int32)
    dst = edge_index[1].astype(jnp.int32)
    pad = EPAD - E
    src_p = jnp.concatenate([src, jnp.zeros((pad,), jnp.int32)])
    dst_p = jnp.concatenate([dst, jnp.full((pad,), N, jnp.int32)])
    src_r = src_p.reshape(NW, NB, B)
    dst_r = dst_p.reshape(NW, NB, B)

    w = jnp.concatenate(
        [Wl1.T, jnp.zeros((D_IN, 16), jnp.float32), Wr1.T], axis=1)
    oh = jnp.zeros((1, 32), jnp.float32).at[0, 16].set(1.0)
    t1, r1 = pl.pallas_call(
        _k1_body,
        out_shape=[
            jax.ShapeDtypeStruct((N, 32), jnp.float32),
            jax.ShapeDtypeStruct((N, 16), jnp.float32),
        ],
    )(x, w, oh)

    acc1 = _seg_sum32(t1, src_r, dst_r)

    h, dinv = pl.pallas_call(
        _k2_body,
        out_shape=[
            jax.ShapeDtypeStruct((N, 16), jnp.float32),
            jax.ShapeDtypeStruct((N, 1), jnp.float32),
        ],
    )(acc1, r1, b1.reshape(1, 16))

    acc2 = _seg_sum16(h, src_r, dst_r)

    w2 = jnp.concatenate([Wl2.T, Wr2.T], axis=0)
    out = pl.pallas_call(
        _k3_body,
        out_shape=jax.ShapeDtypeStruct((N, D_OUT), jnp.float32),
    )(acc2, h, dinv, w2, b2.reshape(1, D_OUT))
    return out


# speedup vs baseline: 22.4379x; 1.6572x over previous
import functools

import jax
import jax.numpy as jnp
from jax import lax
from jax.experimental import pallas as pl
from jax.experimental.pallas import tpu as pltpu
from jax.experimental.pallas import tpu_sc as plsc

N = 10000
E = 320000
D_IN = 128
D_HID = 16
D_OUT = 40

NC = 2
NS = 16
NW = NC * NS
B = 128
NBUF = 8
PIPE = 4
NB = 80
EPW = NB * B
EPAD = NW * EPW
NPAD = 10112
RPT = NPAD // NS


def _make_seg_sum(F: int):
    mesh = plsc.VectorSubcoreMesh(core_axis_name="c", subcore_axis_name="s")

    @functools.partial(
        pl.kernel,
        out_type=jax.ShapeDtypeStruct((NC, NPAD, F), jnp.float32),
        mesh=mesh,
        scratch_types=[
            pltpu.VMEM((NB, B), jnp.int32),
            pltpu.VMEM((NB, B), jnp.int32),
            [pltpu.VMEM((B, F), jnp.float32)] * NBUF,
            pltpu.VMEM((RPT, F), jnp.float32),
            pltpu.VMEM_SHARED((NPAD, F), jnp.float32),
            pltpu.VMEM_SHARED((NPAD, F), jnp.float32),
            [pltpu.SemaphoreType.DMA] * NBUF,
            [pltpu.SemaphoreType.DMA] * NBUF,
        ],
        compiler_params=pltpu.CompilerParams(use_tc_tiling_on_sc=False),
    )
    def seg_sum(table, src, dst, out, src_v, dst_v, rows, zer_v, acc,
                tab_sh, gsem, ssem):
        c = lax.axis_index("c")
        s = lax.axis_index("s")
        wid = c * NS + s

        pltpu.sync_copy(table.at[pl.ds(s * RPT, RPT)],
                        tab_sh.at[pl.ds(s * RPT, RPT)])

        def zero_body(i, _):
            for j in range(F // 16):
                zer_v[i, pl.ds(j * 16, 16)] = jnp.zeros((16,), jnp.float32)
            return 0

        lax.fori_loop(0, RPT, zero_body, 0)
        pltpu.sync_copy(zer_v, acc.at[pl.ds(s * RPT, RPT)])

        pltpu.sync_copy(src.at[wid], src_v)
        pltpu.sync_copy(dst.at[wid], dst_v)
        plsc.subcore_barrier()

        for b in range(PIPE):
            pltpu.async_copy(tab_sh.at[src_v.at[b]], rows[b], gsem[b])

        def group(g, _):
            j0 = g * NBUF
            for b in range(NBUF):
                j = j0 + b
                pltpu.make_async_copy(
                    tab_sh.at[src_v.at[j]], rows[b], gsem[b]).wait()
                pltpu.async_copy(rows[b], acc.at[dst_v.at[j]], ssem[b],
                                 add=True)
                b2 = (b + PIPE) % NBUF
                jn = j + PIPE
                prev = jn - NBUF

                @pl.when(jn < NB)
                def _():
                    @pl.when(prev >= 0)
                    def _():
                        pltpu.make_async_copy(
                            rows[b2], acc.at[dst_v.at[prev]],
                            ssem[b2]).wait()
                    pltpu.async_copy(tab_sh.at[src_v.at[jn]], rows[b2],
                                     gsem[b2])
            return 0

        lax.fori_loop(0, NB // NBUF, group, 0)
        for b in range(NBUF):
            pltpu.make_async_copy(
                rows[b], acc.at[dst_v.at[NB - NBUF + b]], ssem[b]).wait()
        plsc.subcore_barrier()

        pltpu.sync_copy(acc.at[pl.ds(s * RPT, RPT)],
                        out.at[c, pl.ds(s * RPT, RPT)])

    return seg_sum


_seg_sum32 = _make_seg_sum(32)
_seg_sum16 = _make_seg_sum(16)


def _k1_body(x_ref, w_ref, oh_ref, t1_ref, r1_ref):
    pc = jnp.dot(x_ref[:, :], w_ref[:, :], preferred_element_type=jnp.float32)
    t1_ref[0:N, :] = pc[:, 0:32] + oh_ref[:, :]
    t1_ref[N:NPAD, :] = jnp.zeros((NPAD - N, 32), jnp.float32)
    r1_ref[:, :] = pc[:, 32:48]


def _k2_body(acc_ref, r1_ref, b1_ref, h_ref, dinv_ref):
    s1 = acc_ref[0, :N, 0:16] + acc_ref[1, :N, 0:16]
    deg = acc_ref[0, :N, 16:17] + acc_ref[1, :N, 16:17]
    dinv = 1.0 / jnp.maximum(deg, 1.0)
    h_ref[0:N, :] = jnp.maximum(s1 * dinv + b1_ref[:, :] + r1_ref[:, :], 0.0)
    h_ref[N:NPAD, :] = jnp.zeros((NPAD - N, 16), jnp.float32)
    dinv_ref[:, :] = dinv


def _k3_body(acc_ref, h_ref, dinv_ref, w2_ref, b2_ref, out_ref):
    s2 = acc_ref[0, :N, :] + acc_ref[1, :N, :]
    a = s2 * dinv_ref[:, :]
    cat = jnp.concatenate([a, h_ref[0:N, :]], axis=1)
    z = jnp.dot(cat, w2_ref[:, :], preferred_element_type=jnp.float32)
    z = z + b2_ref[:, :]
    m = jnp.max(z, axis=1, keepdims=True)
    e = jnp.exp(z - m)
    lse = jnp.log(jnp.sum(e, axis=1, keepdims=True))
    out_ref[:, :] = z - m - lse


def kernel(x, edge_index, Wl1, Wr1, b1, Wl2, Wr2, b2):
    src = edge_index[0].astype(jnp.int32)
    dst = edge_index[1].astype(jnp.int32)
    pad = EPAD - E
    src_p = jnp.concatenate([src, jnp.zeros((pad,), jnp.int32)])
    dst_p = jnp.concatenate([dst, jnp.full((pad,), N, jnp.int32)])
    src_r = src_p.reshape(NW, NB, B)
    dst_r = dst_p.reshape(NW, NB, B)

    w = jnp.concatenate(
        [Wl1.T, jnp.zeros((D_IN, 16), jnp.float32), Wr1.T], axis=1)
    oh = jnp.zeros((1, 32), jnp.float32).at[0, 16].set(1.0)
    t1, r1 = pl.pallas_call(
        _k1_body,
        out_shape=[
            jax.ShapeDtypeStruct((NPAD, 32), jnp.float32),
            jax.ShapeDtypeStruct((N, 16), jnp.float32),
        ],
    )(x, w, oh)

    acc1 = _seg_sum32(t1, src_r, dst_r)

    h, dinv = pl.pallas_call(
        _k2_body,
        out_shape=[
            jax.ShapeDtypeStruct((NPAD, 16), jnp.float32),
            jax.ShapeDtypeStruct((N, 1), jnp.float32),
        ],
    )(acc1, r1, b1.reshape(1, 16))

    acc2 = _seg_sum16(h, src_r, dst_r)

    w2 = jnp.concatenate([Wl2.T, Wr2.T], axis=0)
    out = pl.pallas_call(
        _k3_body,
        out_shape=jax.ShapeDtypeStruct((N, D_OUT), jnp.float32),
    )(acc2, h, dinv, w2, b2.reshape(1, D_OUT))
    return out


# speedup vs baseline: 30.3130x; 1.3510x over previous
import functools

import jax
import jax.numpy as jnp
from jax import lax
from jax.experimental import pallas as pl
from jax.experimental.pallas import tpu as pltpu
from jax.experimental.pallas import tpu_sc as plsc

N = 10000
E = 320000
D_IN = 128
D_HID = 16
D_OUT = 40

NC = 2
NS = 16
NW = NC * NS
B = 128
NBAT = E // B
NB = NBAT // NW
NXTRA = NBAT - NB * NW
NBUF = 6
PIPE = 3
NPAD = 10112
RPT = NPAD // NS


def _seg_sum_pipeline(eidx, tab_sh, acc, src_v, dst_v, rows, srcx, dstx,
                      gsem, ssem, wid):
    pltpu.sync_copy(eidx.at[0, pl.ds(wid * NB, NB)], src_v)
    pltpu.sync_copy(eidx.at[1, pl.ds(wid * NB, NB)], dst_v)

    for b in range(PIPE):
        pltpu.async_copy(tab_sh.at[src_v.at[b]], rows[b], gsem[b])

    def group(g, _):
        j0 = g * NBUF
        for b in range(NBUF):
            j = j0 + b
            pltpu.make_async_copy(
                tab_sh.at[src_v.at[j]], rows[b], gsem[b]).wait()
            pltpu.async_copy(rows[b], acc.at[dst_v.at[j]], ssem[b],
                             add=True)
            b2 = (b + PIPE) % NBUF
            jn = j + PIPE
            prev = jn - NBUF

            @pl.when(jn < NB)
            def _():
                @pl.when(prev >= 0)
                def _():
                    pltpu.make_async_copy(
                        rows[b2], acc.at[dst_v.at[prev]], ssem[b2]).wait()
                pltpu.async_copy(tab_sh.at[src_v.at[jn]], rows[b2],
                                 gsem[b2])
        return 0

    lax.fori_loop(0, NB // NBUF, group, 0)
    for b in range(NBUF):
        pltpu.make_async_copy(
            rows[b], acc.at[dst_v.at[NB - NBUF + b]], ssem[b]).wait()

    @pl.when(wid < NXTRA)
    def _():
        pltpu.sync_copy(eidx.at[0, pl.ds(NB * NW + wid, 1)], srcx)
        pltpu.sync_copy(eidx.at[1, pl.ds(NB * NW + wid, 1)], dstx)
        pltpu.async_copy(tab_sh.at[srcx.at[0]], rows[0], gsem[0]).wait()
        pltpu.sync_copy(rows[0], acc.at[dstx.at[0]], add=True)


_MESH = plsc.VectorSubcoreMesh(core_axis_name="c", subcore_axis_name="s")
_SC_PARAMS = pltpu.CompilerParams(use_tc_tiling_on_sc=False)


@functools.partial(
    pl.kernel,
    out_type=jax.ShapeDtypeStruct((NC, NPAD, 32), jnp.float32),
    mesh=_MESH,
    scratch_types=[
        pltpu.VMEM((NB, B), jnp.int32),
        pltpu.VMEM((NB, B), jnp.int32),
        [pltpu.VMEM((B, 32), jnp.float32)] * NBUF,
        pltpu.VMEM((1, B), jnp.int32),
        pltpu.VMEM((1, B), jnp.int32),
        pltpu.VMEM((RPT, 32), jnp.float32),
        pltpu.VMEM_SHARED((NPAD, 32), jnp.float32),
        pltpu.VMEM_SHARED((NPAD, 32), jnp.float32),
        [pltpu.SemaphoreType.DMA] * NBUF,
        [pltpu.SemaphoreType.DMA] * NBUF,
    ],
    compiler_params=_SC_PARAMS,
)
def _sc_layer1(t1x, eidx, out, src_v, dst_v, rows, srcx, dstx, zer_v,
               acc, tab_sh, gsem, ssem):
    c = lax.axis_index("c")
    s = lax.axis_index("s")
    wid = c * NS + s
    r0 = s * RPT

    pltpu.sync_copy(t1x.at[pl.ds(r0, RPT), pl.ds(0, 32)],
                    tab_sh.at[pl.ds(r0, RPT)])

    def zero_body(i, _):
        for j in range(2):
            zer_v[i, pl.ds(j * 16, 16)] = jnp.zeros((16,), jnp.float32)
        return 0

    lax.fori_loop(0, RPT, zero_body, 0)
    pltpu.sync_copy(zer_v, acc.at[pl.ds(r0, RPT)])
    plsc.subcore_barrier()

    _seg_sum_pipeline(eidx, tab_sh, acc, src_v, dst_v, rows, srcx, dstx,
                      gsem, ssem, wid)
    plsc.subcore_barrier()

    pltpu.sync_copy(acc.at[pl.ds(r0, RPT)], out.at[c, pl.ds(r0, RPT)])


@functools.partial(
    pl.kernel,
    out_type=jax.ShapeDtypeStruct((NPAD, 128), jnp.float32),
    mesh=_MESH,
    scratch_types=[
        pltpu.VMEM((NB, B), jnp.int32),
        pltpu.VMEM((NB, B), jnp.int32),
        [pltpu.VMEM((B, 16), jnp.float32)] * NBUF,
        pltpu.VMEM((1, B), jnp.int32),
        pltpu.VMEM((1, B), jnp.int32),
        pltpu.VMEM((RPT, 32), jnp.float32),
        pltpu.VMEM((RPT, 32), jnp.float32),
        pltpu.VMEM((RPT, 16), jnp.float32),
        pltpu.VMEM((RPT, 16), jnp.float32),
        pltpu.VMEM((RPT, 16), jnp.float32),
        pltpu.VMEM((16,), jnp.float32),
        pltpu.VMEM_SHARED((NPAD, 16), jnp.float32),
        pltpu.VMEM_SHARED((NPAD, 16), jnp.float32),
        [pltpu.SemaphoreType.DMA] * NBUF,
        [pltpu.SemaphoreType.DMA] * NBUF,
    ],
    compiler_params=_SC_PARAMS,
)
def _sc_layer2(t1x, acc1, b1, eidx, fin, src_v, dst_v, rows, srcx, dstx,
               a0_v, a1_v, r1_v, h_v, di_v, b1_v, acc, tab_sh, gsem, ssem):
    c = lax.axis_index("c")
    s = lax.axis_index("s")
    wid = c * NS + s
    r0 = s * RPT

    pltpu.sync_copy(acc1.at[0, pl.ds(r0, RPT)], a0_v)
    pltpu.sync_copy(acc1.at[1, pl.ds(r0, RPT)], a1_v)
    pltpu.sync_copy(t1x.at[pl.ds(r0, RPT), pl.ds(32, 16)], r1_v)
    pltpu.sync_copy(b1, b1_v)
    b1r = b1_v[...]

    def h_body(i, _):
        srow = a0_v[i, pl.ds(0, 16)] + a1_v[i, pl.ds(0, 16)]
        degv = a0_v[i, pl.ds(16, 16)] + a1_v[i, pl.ds(16, 16)]
        dinvv = 1.0 / jnp.maximum(degv, 1.0)
        hrow = jnp.maximum(srow * dinvv + b1r + r1_v[i, pl.ds(0, 16)], 0.0)
        h_v[i, :] = hrow
        di_v[i, :] = dinvv
        a0_v[i, pl.ds(0, 16)] = jnp.zeros((16,), jnp.float32)
        return 0

    lax.fori_loop(0, RPT, h_body, 0)
    pltpu.sync_copy(h_v, tab_sh.at[pl.ds(r0, RPT)])
    pltpu.sync_copy(a0_v.at[:, pl.ds(0, 16)], acc.at[pl.ds(r0, RPT)])

    @pl.when(c == 0)
    def _():
        pltpu.sync_copy(h_v, fin.at[pl.ds(r0, RPT), pl.ds(32, 16)])
        pltpu.sync_copy(di_v, fin.at[pl.ds(r0, RPT), pl.ds(48, 16)])

    plsc.subcore_barrier()

    _seg_sum_pipeline(eidx, tab_sh, acc, src_v, dst_v, rows, srcx, dstx,
                      gsem, ssem, wid)
    plsc.subcore_barrier()

    pltpu.sync_copy(acc.at[pl.ds(r0, RPT)],
                    fin.at[pl.ds(r0, RPT), pl.ds(c * 16, 16)])


def _k1_body(x_ref, w_ref, oh_ref, t1x_ref):
    pc = jnp.dot(x_ref[:, :], w_ref[:, :], preferred_element_type=jnp.float32)
    t1x_ref[0:N, 0:48] = pc + oh_ref[:, :]
    t1x_ref[N:NPAD, 0:48] = jnp.zeros((NPAD - N, 48), jnp.float32)


def _k3_body(fin_ref, w2_ref, b2_ref, out_ref):
    s2 = fin_ref[0:N, 0:16] + fin_ref[0:N, 16:32]
    h = fin_ref[0:N, 32:48]
    dinv = fin_ref[0:N, 48:49]
    cat = jnp.concatenate([s2 * dinv, h], axis=1)
    z = jnp.dot(cat, w2_ref[:, :], preferred_element_type=jnp.float32)
    z = z + b2_ref[:, :]
    m = jnp.max(z, axis=1, keepdims=True)
    e = jnp.exp(z - m)
    lse = jnp.log(jnp.sum(e, axis=1, keepdims=True))
    out_ref[:, :] = z - m - lse


def kernel(x, edge_index, Wl1, Wr1, b1, Wl2, Wr2, b2):
    eidx = edge_index.astype(jnp.int32).reshape(2, NBAT, B)

    w = jnp.concatenate(
        [Wl1.T, jnp.zeros((D_IN, 16), jnp.float32), Wr1.T], axis=1)
    oh = jnp.concatenate(
        [jnp.zeros((1, 16), jnp.float32), jnp.ones((1, 16), jnp.float32),
         jnp.zeros((1, 16), jnp.float32)], axis=1)
    t1x = pl.pallas_call(
        _k1_body,
        out_shape=jax.ShapeDtypeStruct((NPAD, 128), jnp.float32),
    )(x, w, oh)

    acc1 = _sc_layer1(t1x, eidx)
    fin = _sc_layer2(t1x, acc1, b1, eidx)

    w2 = jnp.concatenate([Wl2.T, Wr2.T], axis=0)
    out = pl.pallas_call(
        _k3_body,
        out_shape=jax.ShapeDtypeStruct((N, D_OUT), jnp.float32),
    )(fin, w2, b2.reshape(1, D_OUT))
    return out


# speedup vs baseline: 32.0149x; 1.0561x over previous
import functools

import jax
import jax.numpy as jnp
from jax import lax
from jax.experimental import pallas as pl
from jax.experimental.pallas import tpu as pltpu
from jax.experimental.pallas import tpu_sc as plsc

N = 10000
E = 320000
D_IN = 128
D_HID = 16
D_OUT = 40

NC = 2
NS = 16
NW = NC * NS
B = 128
NBAT = E // B
NB = NBAT // NW
NXTRA = NBAT - NB * NW
NBUF = 6
PIPE = 3
NPAD = 10112
RPT = NPAD // NS


def _seg_sum_pipeline(eidx, tab_sh, acc, src_v, dst_v, rows, srcx, dstx,
                      gsem, ssem, wid):
    pltpu.sync_copy(eidx.at[0, pl.ds(wid * NB, NB)], src_v)
    pltpu.sync_copy(eidx.at[1, pl.ds(wid * NB, NB)], dst_v)

    for b in range(PIPE):
        pltpu.async_copy(tab_sh.at[src_v.at[b]], rows[b], gsem[b])

    def group(g, _):
        j0 = g * NBUF
        for b in range(NBUF):
            j = j0 + b
            pltpu.make_async_copy(
                tab_sh.at[src_v.at[j]], rows[b], gsem[b]).wait()
            pltpu.async_copy(rows[b], acc.at[dst_v.at[j]], ssem[b],
                             add=True)
            b2 = (b + PIPE) % NBUF
            jn = j + PIPE
            prev = jn - NBUF

            @pl.when(jn < NB)
            def _():
                @pl.when(prev >= 0)
                def _():
                    pltpu.make_async_copy(
                        rows[b2], acc.at[dst_v.at[prev]], ssem[b2]).wait()
                pltpu.async_copy(tab_sh.at[src_v.at[jn]], rows[b2],
                                 gsem[b2])
        return 0

    lax.fori_loop(0, NB // NBUF, group, 0)
    for b in range(NBUF):
        pltpu.make_async_copy(
            rows[b], acc.at[dst_v.at[NB - NBUF + b]], ssem[b]).wait()

    @pl.when(wid < NXTRA)
    def _():
        pltpu.sync_copy(eidx.at[0, pl.ds(NB * NW + wid, 1)], srcx)
        pltpu.sync_copy(eidx.at[1, pl.ds(NB * NW + wid, 1)], dstx)
        pltpu.async_copy(tab_sh.at[srcx.at[0]], rows[0], gsem[0]).wait()
        pltpu.sync_copy(rows[0], acc.at[dstx.at[0]], add=True)


_MESH = plsc.VectorSubcoreMesh(core_axis_name="c", subcore_axis_name="s")
_SC_PARAMS = pltpu.CompilerParams(use_tc_tiling_on_sc=False)


@functools.partial(
    pl.kernel,
    out_type=jax.ShapeDtypeStruct((NC, NPAD, 32), jnp.float32),
    mesh=_MESH,
    scratch_types=[
        pltpu.VMEM((NB, B), jnp.int32),
        pltpu.VMEM((NB, B), jnp.int32),
        [pltpu.VMEM((B, 32), jnp.float32)] * NBUF,
        pltpu.VMEM((1, B), jnp.int32),
        pltpu.VMEM((1, B), jnp.int32),
        pltpu.VMEM((RPT, 32), jnp.float32),
        pltpu.VMEM_SHARED((NPAD, 32), jnp.float32),
        pltpu.VMEM_SHARED((NPAD, 32), jnp.float32),
        [pltpu.SemaphoreType.DMA] * NBUF,
        [pltpu.SemaphoreType.DMA] * NBUF,
    ],
    compiler_params=_SC_PARAMS,
)
def _sc_layer1(t1x, eidx, out, src_v, dst_v, rows, srcx, dstx, zer_v,
               acc, tab_sh, gsem, ssem):
    c = lax.axis_index("c")
    s = lax.axis_index("s")
    wid = c * NS + s
    r0 = s * RPT

    stage = pltpu.async_copy(t1x.at[pl.ds(r0, RPT), pl.ds(0, 32)],
                             tab_sh.at[pl.ds(r0, RPT)], gsem[0])

    def zero_body(i, _):
        for j in range(2):
            zer_v[i, pl.ds(j * 16, 16)] = jnp.zeros((16,), jnp.float32)
        return 0

    lax.fori_loop(0, RPT, zero_body, 0)
    pltpu.sync_copy(zer_v, acc.at[pl.ds(r0, RPT)])
    stage.wait()
    plsc.subcore_barrier()

    _seg_sum_pipeline(eidx, tab_sh, acc, src_v, dst_v, rows, srcx, dstx,
                      gsem, ssem, wid)
    plsc.subcore_barrier()

    pltpu.sync_copy(acc.at[pl.ds(r0, RPT)], out.at[c, pl.ds(r0, RPT)])


@functools.partial(
    pl.kernel,
    out_type=jax.ShapeDtypeStruct((NPAD, 128), jnp.float32),
    mesh=_MESH,
    scratch_types=[
        pltpu.VMEM((NB, B), jnp.int32),
        pltpu.VMEM((NB, B), jnp.int32),
        [pltpu.VMEM((B, 16), jnp.float32)] * NBUF,
        pltpu.VMEM((1, B), jnp.int32),
        pltpu.VMEM((1, B), jnp.int32),
        pltpu.VMEM((RPT, 32), jnp.float32),
        pltpu.VMEM((RPT, 32), jnp.float32),
        pltpu.VMEM((RPT, 16), jnp.float32),
        pltpu.VMEM((RPT, 16), jnp.float32),
        pltpu.VMEM((RPT, 16), jnp.float32),
        pltpu.VMEM((16,), jnp.float32),
        pltpu.VMEM_SHARED((NPAD, 16), jnp.float32),
        pltpu.VMEM_SHARED((NPAD, 16), jnp.float32),
        [pltpu.SemaphoreType.DMA] * NBUF,
        [pltpu.SemaphoreType.DMA] * NBUF,
    ],
    compiler_params=_SC_PARAMS,
)
def _sc_layer2(t1x, acc1, b1, eidx, fin, src_v, dst_v, rows, srcx, dstx,
               a0_v, a1_v, r1_v, h_v, di_v, b1_v, acc, tab_sh, gsem, ssem):
    c = lax.axis_index("c")
    s = lax.axis_index("s")
    wid = c * NS + s
    r0 = s * RPT

    d0 = pltpu.async_copy(acc1.at[0, pl.ds(r0, RPT)], a0_v, gsem[0])
    d1 = pltpu.async_copy(acc1.at[1, pl.ds(r0, RPT)], a1_v, gsem[1])
    d2 = pltpu.async_copy(t1x.at[pl.ds(r0, RPT), pl.ds(32, 16)], r1_v,
                          gsem[2])
    d3 = pltpu.async_copy(b1, b1_v, gsem[3])
    d0.wait()
    d1.wait()
    d2.wait()
    d3.wait()
    b1r = b1_v[...]

    def h_row(i):
        srow = a0_v[i, pl.ds(0, 16)] + a1_v[i, pl.ds(0, 16)]
        degv = a0_v[i, pl.ds(16, 16)] + a1_v[i, pl.ds(16, 16)]
        dinvv = 1.0 / jnp.maximum(degv, 1.0)
        hrow = jnp.maximum(srow * dinvv + b1r + r1_v[i, pl.ds(0, 16)], 0.0)
        h_v[i, :] = hrow
        di_v[i, :] = dinvv
        a0_v[i, pl.ds(0, 16)] = jnp.zeros((16,), jnp.float32)

    def h_body(k, _):
        h_row(2 * k)
        h_row(2 * k + 1)
        return 0

    lax.fori_loop(0, RPT // 2, h_body, 0)
    p0 = pltpu.async_copy(h_v, tab_sh.at[pl.ds(r0, RPT)], gsem[0])
    p1 = pltpu.async_copy(a0_v.at[:, pl.ds(0, 16)], acc.at[pl.ds(r0, RPT)],
                          gsem[1])

    @pl.when(c == 0)
    def _():
        pltpu.async_copy(h_v, fin.at[pl.ds(r0, RPT), pl.ds(32, 16)],
                         ssem[0])
        pltpu.async_copy(di_v, fin.at[pl.ds(r0, RPT), pl.ds(48, 16)],
                         ssem[1])
        pltpu.make_async_copy(h_v, fin.at[pl.ds(r0, RPT), pl.ds(32, 16)],
                              ssem[0]).wait()
        pltpu.make_async_copy(di_v, fin.at[pl.ds(r0, RPT), pl.ds(48, 16)],
                              ssem[1]).wait()

    p0.wait()
    p1.wait()
    plsc.subcore_barrier()

    _seg_sum_pipeline(eidx, tab_sh, acc, src_v, dst_v, rows, srcx, dstx,
                      gsem, ssem, wid)
    plsc.subcore_barrier()

    pltpu.sync_copy(acc.at[pl.ds(r0, RPT)],
                    fin.at[pl.ds(r0, RPT), pl.ds(c * 16, 16)])


def _k1_body(x_ref, w_ref, oh_ref, t1x_ref):
    pc = jnp.dot(x_ref[:, :], w_ref[:, :], preferred_element_type=jnp.float32)
    t1x_ref[0:N, 0:48] = pc + oh_ref[:, :]
    t1x_ref[N:NPAD, 0:48] = jnp.zeros((NPAD - N, 48), jnp.float32)


def _k3_body(fin_ref, w2_ref, b2_ref, out_ref):
    s2 = fin_ref[0:N, 0:16] + fin_ref[0:N, 16:32]
    h = fin_ref[0:N, 32:48]
    dinv = fin_ref[0:N, 48:49]
    cat = jnp.concatenate([s2 * dinv, h], axis=1)
    z = jnp.dot(cat, w2_ref[:, :], preferred_element_type=jnp.float32)
    z = z + b2_ref[:, :]
    m = jnp.max(z, axis=1, keepdims=True)
    e = jnp.exp(z - m)
    lse = jnp.log(jnp.sum(e, axis=1, keepdims=True))
    out_ref[:, :] = z - m - lse


def kernel(x, edge_index, Wl1, Wr1, b1, Wl2, Wr2, b2):
    eidx = edge_index.astype(jnp.int32).reshape(2, NBAT, B)

    w = jnp.concatenate(
        [Wl1.T, jnp.zeros((D_IN, 16), jnp.float32), Wr1.T], axis=1)
    oh = jnp.concatenate(
        [jnp.zeros((1, 16), jnp.float32), jnp.ones((1, 16), jnp.float32),
         jnp.zeros((1, 16), jnp.float32)], axis=1)
    t1x = pl.pallas_call(
        _k1_body,
        out_shape=jax.ShapeDtypeStruct((NPAD, 128), jnp.float32),
    )(x, w, oh)

    acc1 = _sc_layer1(t1x, eidx)
    fin = _sc_layer2(t1x, acc1, b1, eidx)

    w2 = jnp.concatenate([Wl2.T, Wr2.T], axis=0)
    out = pl.pallas_call(
        _k3_body,
        out_shape=jax.ShapeDtypeStruct((N, D_OUT), jnp.float32),
    )(fin, w2, b2.reshape(1, D_OUT))
    return out
